# baseline (speedup 1.0000x reference)
import functools

import jax
import jax.numpy as jnp
from jax import lax
from jax.experimental import pallas as pl
from jax.experimental.pallas import tpu as pltpu
from jax.experimental.pallas import tpu_sc as plsc

_NUM_CORES = 2
_NUM_SUBCORES = 16
_NUM_WORKERS = _NUM_CORES * _NUM_SUBCORES

_CHUNK = 128


@functools.partial(jax.jit, static_argnums=(1, 2))
def _sc_gather(ids2d, n_ids, dim, table):
  per_w = n_ids // _NUM_WORKERS
  n_chunks = per_w // _CHUNK
  mesh = plsc.VectorSubcoreMesh(core_axis_name="c", subcore_axis_name="s")

  @functools.partial(
      pl.kernel,
      mesh=mesh,
      compiler_params=pltpu.CompilerParams(use_tc_tiling_on_sc=False),
      out_type=jax.ShapeDtypeStruct((n_ids, dim), jnp.float32),
      scratch_types=[
          pltpu.VMEM((n_chunks, _CHUNK), jnp.int32),
          pltpu.VMEM((_CHUNK, dim), jnp.float32),
          pltpu.VMEM((_CHUNK, dim), jnp.float32),
          pltpu.SemaphoreType.DMA,
          pltpu.SemaphoreType.DMA,
      ],
  )
  def k(ids_hbm, table_hbm, out_hbm, idx_v, buf0, buf1, s0, s1):
    wid = lax.axis_index("s") * _NUM_CORES + lax.axis_index("c")
    pltpu.sync_copy(ids_hbm.at[pl.ds(wid * n_chunks, n_chunks)], idx_v)
    base = wid * per_w

    pltpu.async_copy(table_hbm.at[idx_v.at[0]], buf0, s0)
    pltpu.async_copy(table_hbm.at[idx_v.at[1]], buf1, s1)

    def pair(i, carry):
      c0 = 2 * i

      pltpu.make_async_copy(table_hbm.at[idx_v.at[c0]], buf0, s0).wait()
      pltpu.sync_copy(buf0, out_hbm.at[pl.ds(base + c0 * _CHUNK, _CHUNK)])

      @pl.when(c0 + 2 < n_chunks)
      def _():
        pltpu.async_copy(table_hbm.at[idx_v.at[c0 + 2]], buf0, s0)

      c1 = c0 + 1
      pltpu.make_async_copy(table_hbm.at[idx_v.at[c1]], buf1, s1).wait()
      pltpu.sync_copy(buf1, out_hbm.at[pl.ds(base + c1 * _CHUNK, _CHUNK)])

      @pl.when(c1 + 2 < n_chunks)
      def _():
        pltpu.async_copy(table_hbm.at[idx_v.at[c1 + 2]], buf1, s1)

      return carry

    lax.fori_loop(0, n_chunks // 2, pair, 0)

  return k(ids2d, table)


def kernel(input_ids, table):
  bsz, seq = input_ids.shape
  n_ids = bsz * seq
  dim = table.shape[1]
  ids2d = input_ids.astype(jnp.int32).reshape(n_ids // _CHUNK, _CHUNK)
  out = _sc_gather(ids2d, n_ids, dim, table)
  return out.reshape(bsz, seq, dim)


# speedup vs baseline: 1.0397x; 1.0397x over previous
import functools

import jax
import jax.numpy as jnp
from jax import lax
from jax.experimental import pallas as pl
from jax.experimental.pallas import tpu as pltpu
from jax.experimental.pallas import tpu_sc as plsc

_NUM_CORES = 2
_NUM_SUBCORES = 16
_NUM_WORKERS = _NUM_CORES * _NUM_SUBCORES

_CHUNK = 40
_NBUF = 4


@functools.partial(jax.jit, static_argnums=(2, 3, 4))
def _sc_gather(ids2, tab2, bsz, seq, dim):
  b_per_w = bsz // _NUM_WORKERS
  n_chunks = b_per_w * (seq // _CHUNK)
  halves = seq // _CHUNK
  mesh = plsc.VectorSubcoreMesh(core_axis_name="c", subcore_axis_name="s")

  @functools.partial(
      pl.kernel,
      mesh=mesh,
      compiler_params=pltpu.CompilerParams(use_tc_tiling_on_sc=False),
      out_type=jax.ShapeDtypeStruct((bsz, seq, dim), jnp.float32),
      scratch_types=[
          pltpu.VMEM((b_per_w, seq), jnp.int32),
          *[pltpu.VMEM((_CHUNK, dim), jnp.float32) for _ in range(_NBUF)],
          *[pltpu.SemaphoreType.DMA for _ in range(_NBUF)],
      ],
  )
  def k(ids_hbm, tab_hbm, out_hbm, idx_v, *bufs_and_sems):
    bufs = bufs_and_sems[:_NBUF]
    sems = bufs_and_sems[_NBUF:]
    wid = lax.axis_index("s") * _NUM_CORES + lax.axis_index("c")
    b0 = wid * b_per_w
    pltpu.sync_copy(ids_hbm.at[pl.ds(b0, b_per_w)], idx_v)

    def idx_ref(t):
      return idx_v.at[t // halves, pl.ds((t % halves) * _CHUNK, _CHUNK)]

    def start(t, kbuf):
      pltpu.async_copy(tab_hbm.at[idx_ref(t)], bufs[kbuf], sems[kbuf])

    def finish(t, kbuf):
      pltpu.make_async_copy(tab_hbm.at[idx_ref(t)], bufs[kbuf], sems[kbuf]).wait()
      pltpu.sync_copy(
          bufs[kbuf],
          out_hbm.at[b0 + t // halves, pl.ds((t % halves) * _CHUNK, _CHUNK)],
      )

    for kbuf in range(_NBUF):
      start(kbuf, kbuf)

    def body(i, carry):
      for kbuf in range(_NBUF):
        t = _NBUF * i + kbuf
        finish(t, kbuf)

        @pl.when(t + _NBUF < n_chunks)
        def _():
          start(t + _NBUF, kbuf)

      return carry

    lax.fori_loop(0, n_chunks // _NBUF, body, 0)

  return k(ids2, tab2)


def kernel(input_ids, table):
  bsz, seq = input_ids.shape
  vocab, dim = table.shape
  ids2 = input_ids.astype(jnp.int32) * 2
  tab2 = jnp.pad(table, ((0, 0), (0, dim))).reshape(2 * vocab, dim)
  return _sc_gather(ids2, tab2, bsz, seq, dim)
